# Initial kernel scaffold; baseline (speedup 1.0000x reference)
import jax
import jax.numpy as jnp
from jax.experimental import pallas as pl


def kernel(x, table):
    raise NotImplementedError("write your pallas kernel here")


# speedup vs baseline: 1.8715x; 1.8715x over previous
import functools

import jax
import jax.numpy as jnp
from jax import lax
from jax.experimental import pallas as pl
from jax.experimental.pallas import tpu as pltpu
from jax.experimental.pallas import tpu_sc as plsc

NC = 2
NS = 16
NW = NC * NS

CHUNK = 128
NBUF = 4


def _emb_body(n_chunks, d, x_hbm, table_hbm, out_hbm, idx_v, rows_v, gsem, osem):
    wid = lax.axis_index("s") * NC + lax.axis_index("c")
    row0 = wid * n_chunks

    pltpu.sync_copy(x_hbm.at[pl.ds(row0, n_chunks)], idx_v)

    def gather(j, b):
        return pltpu.make_async_copy(
            table_hbm.at[idx_v.at[j]], rows_v.at[b], gsem.at[b])

    def writeback(j, b):
        return pltpu.make_async_copy(
            rows_v.at[b], out_hbm.at[pl.ds((row0 + j) * CHUNK, CHUNK)],
            osem.at[b])

    for b in range(NBUF):
        gather(b, b).start()

    def group_body(g, c):
        base = g * NBUF
        for b in range(NBUF):
            gather(base + b, b).wait()
            writeback(base + b, b).start()
        for b in range(NBUF):
            writeback(base + b, b).wait()
            gather(base + NBUF + b, b).start()
        return c

    ngroups = n_chunks // NBUF
    lax.fori_loop(0, ngroups - 1, group_body, 0)

    base = (ngroups - 1) * NBUF
    for b in range(NBUF):
        gather(base + b, b).wait()
        writeback(base + b, b).start()
    for b in range(NBUF):
        writeback(base + b, b).wait()


def kernel(x, table):
    batch, hist = x.shape
    vocab, d = table.shape
    n = batch * hist
    n_rows = n // CHUNK
    n_chunks = n_rows // NW
    assert n_rows * CHUNK == n and n_chunks * NW == n_rows
    assert n_chunks % NBUF == 0

    xf = x.reshape(n_rows, CHUNK).astype(jnp.int32)
    mesh = plsc.VectorSubcoreMesh(core_axis_name="c", subcore_axis_name="s")
    out = pl.kernel(
        functools.partial(_emb_body, n_chunks, d),
        out_type=jax.ShapeDtypeStruct((n, d), table.dtype),
        mesh=mesh,
        compiler_params=pltpu.CompilerParams(use_tc_tiling_on_sc=False),
        scratch_types=[
            pltpu.VMEM((n_chunks, CHUNK), jnp.int32),
            pltpu.VMEM((NBUF, CHUNK, d), jnp.float32),
            pltpu.SemaphoreType.DMA((NBUF,)),
            pltpu.SemaphoreType.DMA((NBUF,)),
        ],
    )(xf, table)
    return out.reshape(batch, hist, d)


# speedup vs baseline: 1.8857x; 1.0076x over previous
import functools

import jax
import jax.numpy as jnp
from jax import lax
from jax.experimental import pallas as pl
from jax.experimental.pallas import tpu as pltpu
from jax.experimental.pallas import tpu_sc as plsc

NC = 2
NS = 16
NW = NC * NS

CHUNK = 128
NBUF = 8


def _emb_body(n_chunks, d, x_hbm, table_hbm, out_hbm, idx_v, rows_v, gsem, osem):
    wid = lax.axis_index("s") * NC + lax.axis_index("c")
    row0 = wid * n_chunks

    pltpu.sync_copy(x_hbm.at[pl.ds(row0, n_chunks)], idx_v)

    def gather(j, b):
        return pltpu.make_async_copy(
            table_hbm.at[idx_v.at[j]], rows_v.at[b], gsem.at[b])

    def writeback(j, b):
        return pltpu.make_async_copy(
            rows_v.at[b], out_hbm.at[pl.ds((row0 + j) * CHUNK, CHUNK)],
            osem.at[b])

    for b in range(NBUF):
        gather(b, b).start()

    def group_body(g, c):
        base = g * NBUF
        for b in range(NBUF):
            gather(base + b, b).wait()
            writeback(base + b, b).start()
        for b in range(NBUF):
            writeback(base + b, b).wait()
            gather(base + NBUF + b, b).start()
        return c

    ngroups = n_chunks // NBUF
    lax.fori_loop(0, ngroups - 1, group_body, 0)

    base = (ngroups - 1) * NBUF
    for b in range(NBUF):
        gather(base + b, b).wait()
        writeback(base + b, b).start()
    for b in range(NBUF):
        writeback(base + b, b).wait()


def kernel(x, table):
    batch, hist = x.shape
    vocab, d = table.shape
    n = batch * hist
    n_rows = n // CHUNK
    n_chunks = n_rows // NW
    assert n_rows * CHUNK == n and n_chunks * NW == n_rows
    assert n_chunks % NBUF == 0

    xf = x.reshape(n_rows, CHUNK).astype(jnp.int32)
    mesh = plsc.VectorSubcoreMesh(core_axis_name="c", subcore_axis_name="s")
    out = pl.kernel(
        functools.partial(_emb_body, n_chunks, d),
        out_type=jax.ShapeDtypeStruct((n, d), table.dtype),
        mesh=mesh,
        compiler_params=pltpu.CompilerParams(use_tc_tiling_on_sc=False),
        scratch_types=[
            pltpu.VMEM((n_chunks, CHUNK), jnp.int32),
            pltpu.VMEM((NBUF, CHUNK, d), jnp.float32),
            pltpu.SemaphoreType.DMA((NBUF,)),
            pltpu.SemaphoreType.DMA((NBUF,)),
        ],
    )(xf, table)
    return out.reshape(batch, hist, d)
